# baseline (speedup 1.0000x reference)
import functools

import jax
import jax.numpy as jnp
from jax import lax
from jax.experimental import pallas as pl
from jax.experimental.pallas import tpu as pltpu
from jax.experimental.pallas import tpu_sc as plsc

N = 10000
E = 320000
D = 128
NREL = 1000
NQ = 64000
B = 4096
TPAD = 368

NC = 2
NS = 16
NW = NC * NS
L = 16
VPR = D // L

EK = 40
E_PER_W = E // NW
E_CHUNKS = E_PER_W // EK
Q_PER_W = NQ // NW
Q_CHUNKS = Q_PER_W // EK
B_PER_W = B // NW
BK = 64
B_CHUNKS = B_PER_W // BK
NPAD = 10240
N_PER_S = NPAD // NS
ZK = 40
Z_CHUNKS = N_PER_S // ZK

def _mesh():
    return plsc.VectorSubcoreMesh(core_axis_name="c", subcore_axis_name="s",
                                  num_cores=NC, num_subcores=NS)


def _rows_op(dst_ref, k, body):
    def step(i, _):
        for jj in range(VPR):
            dst_ref[i, pl.ds(jj * L, L)] = body(i, jj * L)
        return 0
    lax.fori_loop(0, k, step, 0)


def _sc_agg_body(x0_hbm, tab_hbm, tt_hbm, src_hbm, dst_hbm, ebt_hbm,
                 etime_hbm, qr_hbm, qe_hbm, qedge_hbm,
                 p0_hbm, p1_hbm,
                 ib0, ib1, bt0, bt1,
                 abuf0, abuf1, btbuf0, btbuf1,
                 aggS, semI0, semI1, semR0, semR1,
                 semQ0, semQ1):
    c = lax.axis_index("c")
    s = lax.axis_index("s")
    w = c * NS + s

    set0 = (ib0, bt0, abuf0, btbuf0, semI0, semR0)
    set1 = (ib1, bt1, abuf1, btbuf1, semI1, semR1)

    zero16 = jnp.zeros((L,), jnp.float32)
    _rows_op(abuf0, ZK, lambda i, j: zero16)
    r0 = s * N_PER_S

    def _zc(k, _):
        pltpu.sync_copy(abuf0, aggS.at[pl.ds(r0 + k * ZK, ZK)])
        return 0
    lax.fori_loop(0, Z_CHUNKS, _zc, 0)
    plsc.subcore_barrier()

    e0 = w * E_PER_W

    def _issue_idx(ci, st):
        base = e0 + ci * EK
        pltpu.async_copy(src_hbm.at[pl.ds(base, EK)], st[0].at[0], st[4])
        pltpu.async_copy(dst_hbm.at[pl.ds(base, EK)], st[0].at[1], st[4])
        pltpu.async_copy(ebt_hbm.at[pl.ds(2 * base, 2 * EK)], st[1], st[4])

    def _wait_idx(st):
        pltpu.make_async_copy(src_hbm.at[pl.ds(0, EK)], st[0].at[0], st[4]).wait()
        pltpu.make_async_copy(dst_hbm.at[pl.ds(0, EK)], st[0].at[1], st[4]).wait()
        pltpu.make_async_copy(ebt_hbm.at[pl.ds(0, 2 * EK)], st[1], st[4]).wait()

    def _issue_rows(st):
        pltpu.async_copy(x0_hbm.at[st[0].at[0]], st[2], st[5])
        pltpu.async_copy(tab_hbm.at[st[1]], st[3], st[5])

    def _wait_rows(st):
        pltpu.make_async_copy(x0_hbm.at[st[0].at[0]], st[2], st[5]).wait()
        pltpu.make_async_copy(tab_hbm.at[st[1]], st[3], st[5]).wait()

    def _estep(cur, stS, stT):
        @pl.when(cur + 1 < E_CHUNKS)
        def _():
            _wait_idx(stT)
            _issue_rows(stT)
        _wait_rows(stS)
        a, bt = stS[2], stS[3]
        _rows_op(a, EK, lambda i, j:
                 (a[i, pl.ds(j, L)] - bt[i, pl.ds(j, L)])
                 * bt[EK + i, pl.ds(j, L)])
        pltpu.sync_copy(a, aggS.at[stS[0].at[1]], add=True)

        @pl.when(cur + 2 < E_CHUNKS)
        def _():
            _issue_idx(cur + 2, stS)

    _issue_idx(0, set0)
    _wait_idx(set0)
    _issue_rows(set0)
    _issue_idx(1, set1)

    def _epair(p, _):
        _estep(2 * p, set0, set1)
        _estep(2 * p + 1, set1, set0)
        return 0
    lax.fori_loop(0, E_CHUNKS // 2, _epair, 0)

    q0 = w * Q_PER_W

    def _q_issue_qidx(ci, st, semQ):
        base = q0 + ci * EK
        pltpu.async_copy(qedge_hbm.at[pl.ds(base, EK)], st[0].at[0], semQ)
        pltpu.async_copy(qr_hbm.at[pl.ds(base, EK)], st[0].at[2], semQ)
        pltpu.async_copy(qe_hbm.at[pl.ds(base, EK)], st[0].at[4], semQ)

    def _q_wait_qidx(st, semQ):
        pltpu.make_async_copy(qedge_hbm.at[pl.ds(0, EK)], st[0].at[0], semQ).wait()
        pltpu.make_async_copy(qr_hbm.at[pl.ds(0, EK)], st[0].at[2], semQ).wait()
        pltpu.make_async_copy(qe_hbm.at[pl.ds(0, EK)], st[0].at[4], semQ).wait()

    def _q_issue_scal(st):
        pltpu.async_copy(dst_hbm.at[st[0].at[0]], st[0].at[1], st[4])
        pltpu.async_copy(etime_hbm.at[st[0].at[0]], st[0].at[3], st[4])

    def _q_wait_scal(st):
        pltpu.make_async_copy(dst_hbm.at[st[0].at[0]], st[0].at[1], st[4]).wait()
        pltpu.make_async_copy(etime_hbm.at[st[0].at[0]], st[0].at[3], st[4]).wait()

    def _q_issue_rows(st):
        pltpu.async_copy(x0_hbm.at[st[0].at[4]], st[2], st[5])
        pltpu.async_copy(tab_hbm.at[st[0].at[2]], st[3].at[pl.ds(0, EK)], st[5])
        pltpu.async_copy(tt_hbm.at[st[0].at[3]], st[3].at[pl.ds(EK, EK)], st[5])

    def _q_wait_rows(st):
        pltpu.make_async_copy(x0_hbm.at[st[0].at[4]], st[2], st[5]).wait()
        pltpu.make_async_copy(tab_hbm.at[st[0].at[2]], st[3].at[pl.ds(0, EK)],
                              st[5]).wait()
        pltpu.make_async_copy(tt_hbm.at[st[0].at[3]], st[3].at[pl.ds(EK, EK)],
                              st[5]).wait()

    def _qstep(cur, stS, stT, semQS, semQT):
        @pl.when(cur + 1 < Q_CHUNKS)
        def _():
            _q_wait_qidx(stT, semQT)
            _q_issue_scal(stT)
        _q_wait_rows(stS)

        @pl.when(cur + 1 < Q_CHUNKS)
        def _():
            _q_wait_scal(stT)
            _q_issue_rows(stT)
        a, bt = stS[2], stS[3]
        _rows_op(a, EK, lambda i, j:
                 -(a[i, pl.ds(j, L)] * bt[i, pl.ds(j, L)])
                 * bt[EK + i, pl.ds(j, L)])
        pltpu.sync_copy(a, aggS.at[stS[0].at[1]], add=True)

        @pl.when(cur + 2 < Q_CHUNKS)
        def _():
            _q_issue_qidx(cur + 2, stS, semQS)

    _q_issue_qidx(0, set0, semQ0)
    _q_wait_qidx(set0, semQ0)
    _q_issue_scal(set0)
    _q_wait_scal(set0)
    _q_issue_rows(set0)
    _q_issue_qidx(1, set1, semQ1)

    def _qpair(p, _):
        _qstep(2 * p, set0, set1, semQ0, semQ1)
        _qstep(2 * p + 1, set1, set0, semQ1, semQ0)
        return 0
    lax.fori_loop(0, Q_CHUNKS // 2, _qpair, 0)

    plsc.subcore_barrier()

    def _oc(k, _):
        r = r0 + k * ZK
        pltpu.sync_copy(aggS.at[pl.ds(r, ZK)], abuf0)

        @pl.when(c == 0)
        def _():
            pltpu.sync_copy(abuf0, p0_hbm.at[pl.ds(r, ZK)])

        @pl.when(c == 1)
        def _():
            pltpu.sync_copy(abuf0, p1_hbm.at[pl.ds(r, ZK)])
        return 0
    lax.fori_loop(0, Z_CHUNKS, _oc, 0)


def _sc_agg():
  return pl.kernel(
    _sc_agg_body,
    out_type=(jax.ShapeDtypeStruct((NPAD, D), jnp.float32),
              jax.ShapeDtypeStruct((NPAD, D), jnp.float32)),
    mesh=_mesh(),
    scratch_types=[
        pltpu.VMEM((5, EK), jnp.int32), pltpu.VMEM((5, EK), jnp.int32),
        pltpu.VMEM((2 * EK,), jnp.int32), pltpu.VMEM((2 * EK,), jnp.int32),
        pltpu.VMEM((EK, D), jnp.float32), pltpu.VMEM((EK, D), jnp.float32),
        pltpu.VMEM((2 * EK, D), jnp.float32),
        pltpu.VMEM((2 * EK, D), jnp.float32),
        pltpu.VMEM_SHARED((NPAD, D), jnp.float32),
        pltpu.SemaphoreType.DMA, pltpu.SemaphoreType.DMA,
        pltpu.SemaphoreType.DMA, pltpu.SemaphoreType.DMA,
        pltpu.SemaphoreType.DMA, pltpu.SemaphoreType.DMA,
    ],
  )


def _sc_query_body(p0_hbm, p1_hbm, reltab_hbm, sub_hbm, rel_hbm,
                   subrows_hbm, relrows_hbm,
                   iS, iR, abuf, bbuf, mbuf, sem0, sem1):
    c = lax.axis_index("c")
    s = lax.axis_index("s")
    w = c * NS + s
    b0 = w * B_PER_W

    def _chunk(ci, _):
        base = b0 + ci * BK
        pltpu.sync_copy(sub_hbm.at[pl.ds(base, BK)], iS)
        pltpu.sync_copy(rel_hbm.at[pl.ds(base, BK)], iR)
        cp1 = pltpu.async_copy(p0_hbm.at[iS], abuf, sem0)
        cp2 = pltpu.async_copy(p1_hbm.at[iS], bbuf, sem1)
        cp1.wait()
        cp2.wait()
        _rows_op(mbuf, BK, lambda i, j:
                 abuf[i, pl.ds(j, L)] + bbuf[i, pl.ds(j, L)])
        pltpu.sync_copy(mbuf, subrows_hbm.at[pl.ds(base, BK)])
        cp3 = pltpu.async_copy(reltab_hbm.at[iR], abuf, sem0)
        cp3.wait()
        pltpu.sync_copy(abuf, relrows_hbm.at[pl.ds(base, BK)])
        return 0
    lax.fori_loop(0, B_CHUNKS, _chunk, 0)


def _sc_query():
  return pl.kernel(
    _sc_query_body,
    out_type=(jax.ShapeDtypeStruct((B, D), jnp.float32),
              jax.ShapeDtypeStruct((B, D), jnp.float32)),
    mesh=_mesh(),
    scratch_types=[
        pltpu.VMEM((BK,), jnp.int32), pltpu.VMEM((BK,), jnp.int32),
        pltpu.VMEM((BK, D), jnp.float32), pltpu.VMEM((BK, D), jnp.float32),
        pltpu.VMEM((BK, D), jnp.float32),
        pltpu.SemaphoreType.DMA, pltpu.SemaphoreType.DMA,
    ],
  )


def _tc_time_body(trange_ref, timef_ref, w_ref, phi_ref, tt_ref, temb_ref):
    tt_ref[...] = jnp.cos(trange_ref[...] * w_ref[...] + phi_ref[...])
    temb_ref[...] = jnp.cos(timef_ref[...] * w_ref[...] + phi_ref[...])


def _tc_x_body(p0_ref, p1_ref, w_ref, x_ref):
    acc = p0_ref[...] + p1_ref[...]
    x_ref[...] = jnp.tanh(
        jnp.dot(acc, w_ref[...], preferred_element_type=jnp.float32))


def _tc_q_body(subrows_ref, temb_ref, relrows_ref, w_ref, wpt_ref, wpb_ref,
               bproj_ref, wrel_ref, sub_emb_ref, rel_emb_ref):
    sx = jnp.tanh(jnp.dot(subrows_ref[...], w_ref[...],
                          preferred_element_type=jnp.float32))
    sub_emb_ref[...] = (
        jnp.dot(sx, wpt_ref[...], preferred_element_type=jnp.float32)
        + jnp.dot(temb_ref[...], wpb_ref[...],
                  preferred_element_type=jnp.float32)
        + bproj_ref[...])
    rel_emb_ref[...] = jnp.dot(relrows_ref[...], wrel_ref[...],
                               preferred_element_type=jnp.float32)


def kernel(x0, init_rel, W, w_rel, basis_freq, phase, W_proj, b_proj,
           edge_index, edge_type, edge_time, quals, sub, rel, time):
    src = edge_index[0]
    dst = edge_index[1]
    qr, qe, qedge = quals[0], quals[1], quals[2]

    trange = jnp.arange(TPAD, dtype=jnp.float32)[:, None]
    timef = time.astype(jnp.float32)[:, None]
    wrow = basis_freq[None, :]
    phirow = phase[None, :]

    tt, temb = pl.pallas_call(
        _tc_time_body,
        out_shape=(jax.ShapeDtypeStruct((TPAD, D), jnp.float32),
                   jax.ShapeDtypeStruct((B, D), jnp.float32)),
    )(trange, timef, wrow, phirow)

    tab = jnp.concatenate([init_rel, tt], axis=0)
    ebt = jnp.stack([edge_type.reshape(E // EK, EK),
                     (edge_time + NREL).reshape(E // EK, EK)],
                    axis=1).reshape(-1)
    p0, p1 = _sc_agg()(x0, tab, tt, src, dst, ebt, edge_time,
                       qr, qe, qedge)

    subrows, relrows = _sc_query()(p0, p1, init_rel, sub, rel)

    bs = 1024
    x = pl.pallas_call(
        _tc_x_body,
        grid=(pl.cdiv(N, bs),),
        in_specs=[pl.BlockSpec((bs, D), lambda i: (i, 0)),
                  pl.BlockSpec((bs, D), lambda i: (i, 0)),
                  pl.BlockSpec((D, D), lambda i: (0, 0))],
        out_specs=pl.BlockSpec((bs, D), lambda i: (i, 0)),
        out_shape=jax.ShapeDtypeStruct((N, D), jnp.float32),
    )(p0, p1, W)

    sub_emb, rel_emb = pl.pallas_call(
        _tc_q_body,
        out_shape=(jax.ShapeDtypeStruct((B, D), jnp.float32),
                   jax.ShapeDtypeStruct((B, D), jnp.float32)),
    )(subrows, temb, relrows, W, W_proj[:D], W_proj[D:], b_proj[None, :],
      w_rel)

    return sub_emb, rel_emb, x, temb


# speedup vs baseline: 1.1548x; 1.1548x over previous
import functools

import jax
import jax.numpy as jnp
from jax import lax
from jax.experimental import pallas as pl
from jax.experimental.pallas import tpu as pltpu
from jax.experimental.pallas import tpu_sc as plsc

N = 10000
E = 320000
D = 128
NREL = 1000
NQ = 64000
B = 4096
TPAD = 368

NC = 2
NS = 16
NW = NC * NS
L = 16
VPR = D // L

EK = 40
E_PER_W = E // NW
E_CHUNKS = E_PER_W // EK
Q_PER_W = NQ // NW
Q_CHUNKS = Q_PER_W // EK
B_PER_W = B // NW
BK = 64
B_CHUNKS = B_PER_W // BK
NPAD = 10240
N_PER_S = NPAD // NS
ZK = 40
Z_CHUNKS = N_PER_S // ZK

def _mesh():
    return plsc.VectorSubcoreMesh(core_axis_name="c", subcore_axis_name="s",
                                  num_cores=NC, num_subcores=NS)


def _rows_op(dst_ref, k, body):
    def step(i, _):
        for jj in range(VPR):
            dst_ref[i, pl.ds(jj * L, L)] = body(i, jj * L)
        return 0
    lax.fori_loop(0, k, step, 0)


def _sc_agg_body(x0_hbm, rel_hbm, tt_hbm, src_hbm, dst_hbm, et_hbm, etime_hbm,
                 qr_hbm, qe_hbm, qedge_hbm,
                 p0_hbm, p1_hbm,
                 ib0, ib1, ib2,
                 abuf0, bbuf0, tbuf0, abuf1, bbuf1, tbuf1, abuf2, bbuf2, tbuf2,
                 aggS, semI0, semI1, semI2, semR0, semR1, semR2,
                 semQ0, semQ1):
    c = lax.axis_index("c")
    s = lax.axis_index("s")
    w = c * NS + s

    set0 = (ib0.at[0], ib0.at[1], ib0.at[2], ib0.at[3], abuf0, bbuf0, tbuf0,
            abuf0, semI0, semR0, ib0.at[4], semQ0)
    set1 = (ib1.at[0], ib1.at[1], ib1.at[2], ib1.at[3], abuf1, bbuf1, tbuf1,
            abuf1, semI1, semR1, ib1.at[4], semQ1)
    set2 = (ib2.at[0], ib2.at[1], ib2.at[2], ib2.at[3], abuf2, bbuf2, tbuf2,
            abuf2, semI2, semR2, ib2.at[4], semQ0)

    zero16 = jnp.zeros((L,), jnp.float32)
    _rows_op(abuf0, ZK, lambda i, j: zero16)
    r0 = s * N_PER_S

    def _zc(k, _):
        pltpu.sync_copy(abuf0, aggS.at[pl.ds(r0 + k * ZK, ZK)])
        return 0
    lax.fori_loop(0, Z_CHUNKS, _zc, 0)
    plsc.subcore_barrier()

    e0 = w * E_PER_W

    def _issue_idx(ci, st):
        i0, i1, i2, i3 = st[0], st[1], st[2], st[3]
        base = e0 + ci * EK
        pltpu.async_copy(src_hbm.at[pl.ds(base, EK)], i0, st[8])
        pltpu.async_copy(et_hbm.at[pl.ds(base, EK)], i1, st[8])
        pltpu.async_copy(etime_hbm.at[pl.ds(base, EK)], i2, st[8])
        pltpu.async_copy(dst_hbm.at[pl.ds(base, EK)], i3, st[8])

    def _wait_idx(st):
        pltpu.make_async_copy(src_hbm.at[pl.ds(0, EK)], st[0], st[8]).wait()
        pltpu.make_async_copy(et_hbm.at[pl.ds(0, EK)], st[1], st[8]).wait()
        pltpu.make_async_copy(etime_hbm.at[pl.ds(0, EK)], st[2], st[8]).wait()
        pltpu.make_async_copy(dst_hbm.at[pl.ds(0, EK)], st[3], st[8]).wait()

    def _issue_rows(st):
        pltpu.async_copy(x0_hbm.at[st[0]], st[4], st[9])
        pltpu.async_copy(rel_hbm.at[st[1]], st[5], st[9])
        pltpu.async_copy(tt_hbm.at[st[2]], st[6], st[9])

    def _wait_rows(st):
        pltpu.make_async_copy(x0_hbm.at[st[0]], st[4], st[9]).wait()
        pltpu.make_async_copy(rel_hbm.at[st[1]], st[5], st[9]).wait()
        pltpu.make_async_copy(tt_hbm.at[st[2]], st[6], st[9]).wait()

    def _estep(cur, stS, stT2):
        @pl.when(cur + 2 < E_CHUNKS)
        def _():
            _wait_idx(stT2)
            _issue_rows(stT2)
        _wait_rows(stS)
        a, b, t = stS[4], stS[5], stS[6]
        _rows_op(a, EK, lambda i, j:
                 (a[i, pl.ds(j, L)] - b[i, pl.ds(j, L)]) * t[i, pl.ds(j, L)])
        pltpu.sync_copy(a, aggS.at[stS[3]], add=True)

        @pl.when(cur + 3 < E_CHUNKS)
        def _():
            _issue_idx(cur + 3, stS)

    _issue_idx(0, set0)
    _issue_idx(1, set1)
    _wait_idx(set0)
    _issue_rows(set0)
    _wait_idx(set1)
    _issue_rows(set1)
    _issue_idx(2, set2)

    def _etriple(p, _):
        _estep(3 * p, set0, set2)
        _estep(3 * p + 1, set1, set0)
        _estep(3 * p + 2, set2, set1)
        return 0
    lax.fori_loop(0, E_CHUNKS // 3, _etriple, 0)
    _estep(E_CHUNKS - 1, set0, set2)

    q0 = w * Q_PER_W
    qseta = set0
    qsetb = set1

    def _q_issue_qidx(ci, st):
        base = q0 + ci * EK
        pltpu.async_copy(qedge_hbm.at[pl.ds(base, EK)], st[0], st[11])
        pltpu.async_copy(qr_hbm.at[pl.ds(base, EK)], st[1], st[11])
        pltpu.async_copy(qe_hbm.at[pl.ds(base, EK)], st[10], st[11])

    def _q_wait_qidx(st):
        pltpu.make_async_copy(qedge_hbm.at[pl.ds(0, EK)], st[0], st[11]).wait()
        pltpu.make_async_copy(qr_hbm.at[pl.ds(0, EK)], st[1], st[11]).wait()
        pltpu.make_async_copy(qe_hbm.at[pl.ds(0, EK)], st[10], st[11]).wait()

    def _q_issue_scal(st):
        pltpu.async_copy(dst_hbm.at[st[0]], st[3], st[8])
        pltpu.async_copy(etime_hbm.at[st[0]], st[2], st[8])

    def _q_wait_scal(st):
        pltpu.make_async_copy(dst_hbm.at[st[0]], st[3], st[8]).wait()
        pltpu.make_async_copy(etime_hbm.at[st[0]], st[2], st[8]).wait()

    def _q_issue_rows(st):
        pltpu.async_copy(x0_hbm.at[st[10]], st[4], st[9])
        pltpu.async_copy(rel_hbm.at[st[1]], st[5], st[9])
        pltpu.async_copy(tt_hbm.at[st[2]], st[6], st[9])

    def _q_wait_rows(st):
        pltpu.make_async_copy(x0_hbm.at[st[10]], st[4], st[9]).wait()
        pltpu.make_async_copy(rel_hbm.at[st[1]], st[5], st[9]).wait()
        pltpu.make_async_copy(tt_hbm.at[st[2]], st[6], st[9]).wait()

    def _qstep(cur, stS, stT):
        @pl.when(cur + 1 < Q_CHUNKS)
        def _():
            _q_wait_qidx(stT)
            _q_issue_scal(stT)
        _q_wait_rows(stS)

        @pl.when(cur + 1 < Q_CHUNKS)
        def _():
            _q_wait_scal(stT)
            _q_issue_rows(stT)
        a, b, t, m = stS[4], stS[5], stS[6], stS[7]
        _rows_op(m, EK, lambda i, j:
                 -(a[i, pl.ds(j, L)] * b[i, pl.ds(j, L)]) * t[i, pl.ds(j, L)])
        pltpu.sync_copy(m, aggS.at[stS[3]], add=True)

        @pl.when(cur + 2 < Q_CHUNKS)
        def _():
            _q_issue_qidx(cur + 2, stS)

    _q_issue_qidx(0, qseta)
    _q_wait_qidx(qseta)
    _q_issue_scal(qseta)
    _q_wait_scal(qseta)
    _q_issue_rows(qseta)
    _q_issue_qidx(1, qsetb)

    def _qpair(p, _):
        _qstep(2 * p, qseta, qsetb)
        _qstep(2 * p + 1, qsetb, qseta)
        return 0
    lax.fori_loop(0, Q_CHUNKS // 2, _qpair, 0)

    plsc.subcore_barrier()

    def _oc(k, _):
        r = r0 + k * ZK
        pltpu.sync_copy(aggS.at[pl.ds(r, ZK)], abuf0)

        @pl.when(c == 0)
        def _():
            pltpu.sync_copy(abuf0, p0_hbm.at[pl.ds(r, ZK)])

        @pl.when(c == 1)
        def _():
            pltpu.sync_copy(abuf0, p1_hbm.at[pl.ds(r, ZK)])
        return 0
    lax.fori_loop(0, Z_CHUNKS, _oc, 0)


def _sc_agg():
  return pl.kernel(
    _sc_agg_body,
    out_type=(jax.ShapeDtypeStruct((NPAD, D), jnp.float32),
              jax.ShapeDtypeStruct((NPAD, D), jnp.float32)),
    mesh=_mesh(),
    scratch_types=[
        pltpu.VMEM((5, EK), jnp.int32), pltpu.VMEM((5, EK), jnp.int32),
        pltpu.VMEM((5, EK), jnp.int32),
        pltpu.VMEM((EK, D), jnp.float32), pltpu.VMEM((EK, D), jnp.float32),
        pltpu.VMEM((EK, D), jnp.float32), pltpu.VMEM((EK, D), jnp.float32),
        pltpu.VMEM((EK, D), jnp.float32), pltpu.VMEM((EK, D), jnp.float32),
        pltpu.VMEM((EK, D), jnp.float32), pltpu.VMEM((EK, D), jnp.float32),
        pltpu.VMEM((EK, D), jnp.float32),
        pltpu.VMEM_SHARED((NPAD, D), jnp.float32),
        pltpu.SemaphoreType.DMA, pltpu.SemaphoreType.DMA,
        pltpu.SemaphoreType.DMA, pltpu.SemaphoreType.DMA,
        pltpu.SemaphoreType.DMA, pltpu.SemaphoreType.DMA,
        pltpu.SemaphoreType.DMA, pltpu.SemaphoreType.DMA,
    ],
  )


def _sc_query_body(p0_hbm, p1_hbm, reltab_hbm, sub_hbm, rel_hbm,
                   subrows_hbm, relrows_hbm,
                   iS, iR, abuf, bbuf, mbuf, sem0, sem1):
    c = lax.axis_index("c")
    s = lax.axis_index("s")
    w = c * NS + s
    b0 = w * B_PER_W

    def _chunk(ci, _):
        base = b0 + ci * BK
        pltpu.sync_copy(sub_hbm.at[pl.ds(base, BK)], iS)
        pltpu.sync_copy(rel_hbm.at[pl.ds(base, BK)], iR)
        cp1 = pltpu.async_copy(p0_hbm.at[iS], abuf, sem0)
        cp2 = pltpu.async_copy(p1_hbm.at[iS], bbuf, sem1)
        cp1.wait()
        cp2.wait()
        _rows_op(mbuf, BK, lambda i, j:
                 abuf[i, pl.ds(j, L)] + bbuf[i, pl.ds(j, L)])
        pltpu.sync_copy(mbuf, subrows_hbm.at[pl.ds(base, BK)])
        cp3 = pltpu.async_copy(reltab_hbm.at[iR], abuf, sem0)
        cp3.wait()
        pltpu.sync_copy(abuf, relrows_hbm.at[pl.ds(base, BK)])
        return 0
    lax.fori_loop(0, B_CHUNKS, _chunk, 0)


def _sc_query():
  return pl.kernel(
    _sc_query_body,
    out_type=(jax.ShapeDtypeStruct((B, D), jnp.float32),
              jax.ShapeDtypeStruct((B, D), jnp.float32)),
    mesh=_mesh(),
    scratch_types=[
        pltpu.VMEM((BK,), jnp.int32), pltpu.VMEM((BK,), jnp.int32),
        pltpu.VMEM((BK, D), jnp.float32), pltpu.VMEM((BK, D), jnp.float32),
        pltpu.VMEM((BK, D), jnp.float32),
        pltpu.SemaphoreType.DMA, pltpu.SemaphoreType.DMA,
    ],
  )


def _tc_time_body(trange_ref, timef_ref, w_ref, phi_ref, tt_ref, temb_ref):
    tt_ref[...] = jnp.cos(trange_ref[...] * w_ref[...] + phi_ref[...])
    temb_ref[...] = jnp.cos(timef_ref[...] * w_ref[...] + phi_ref[...])


def _tc_x_body(p0_ref, p1_ref, w_ref, x_ref):
    acc = p0_ref[...] + p1_ref[...]
    x_ref[...] = jnp.tanh(
        jnp.dot(acc, w_ref[...], preferred_element_type=jnp.float32))


def _tc_q_body(subrows_ref, temb_ref, relrows_ref, w_ref, wpt_ref, wpb_ref,
               bproj_ref, wrel_ref, sub_emb_ref, rel_emb_ref):
    sx = jnp.tanh(jnp.dot(subrows_ref[...], w_ref[...],
                          preferred_element_type=jnp.float32))
    sub_emb_ref[...] = (
        jnp.dot(sx, wpt_ref[...], preferred_element_type=jnp.float32)
        + jnp.dot(temb_ref[...], wpb_ref[...],
                  preferred_element_type=jnp.float32)
        + bproj_ref[...])
    rel_emb_ref[...] = jnp.dot(relrows_ref[...], wrel_ref[...],
                               preferred_element_type=jnp.float32)


def kernel(x0, init_rel, W, w_rel, basis_freq, phase, W_proj, b_proj,
           edge_index, edge_type, edge_time, quals, sub, rel, time):
    src = edge_index[0]
    dst = edge_index[1]
    qr, qe, qedge = quals[0], quals[1], quals[2]

    trange = jnp.arange(TPAD, dtype=jnp.float32)[:, None]
    timef = time.astype(jnp.float32)[:, None]
    wrow = basis_freq[None, :]
    phirow = phase[None, :]

    tt, temb = pl.pallas_call(
        _tc_time_body,
        out_shape=(jax.ShapeDtypeStruct((TPAD, D), jnp.float32),
                   jax.ShapeDtypeStruct((B, D), jnp.float32)),
    )(trange, timef, wrow, phirow)

    p0, p1 = _sc_agg()(x0, init_rel, tt, src, dst, edge_type, edge_time,
                       qr, qe, qedge)

    subrows, relrows = _sc_query()(p0, p1, init_rel, sub, rel)

    bs = 1024
    x = pl.pallas_call(
        _tc_x_body,
        grid=(pl.cdiv(N, bs),),
        in_specs=[pl.BlockSpec((bs, D), lambda i: (i, 0)),
                  pl.BlockSpec((bs, D), lambda i: (i, 0)),
                  pl.BlockSpec((D, D), lambda i: (0, 0))],
        out_specs=pl.BlockSpec((bs, D), lambda i: (i, 0)),
        out_shape=jax.ShapeDtypeStruct((N, D), jnp.float32),
    )(p0, p1, W)

    sub_emb, rel_emb = pl.pallas_call(
        _tc_q_body,
        out_shape=(jax.ShapeDtypeStruct((B, D), jnp.float32),
                   jax.ShapeDtypeStruct((B, D), jnp.float32)),
    )(subrows, temb, relrows, W, W_proj[:D], W_proj[D:], b_proj[None, :],
      w_rel)

    return sub_emb, rel_emb, x, temb
